# baseline (speedup 1.0000x reference)
import functools

import jax
import jax.numpy as jnp
from jax import lax
from jax.experimental import pallas as pl
from jax.experimental.pallas import tpu as pltpu
from jax.experimental.pallas import tpu_sc as plsc

_TN = 256
_SW = 128
_G = _SW // 8


def _tree_min(vs):
    while len(vs) > 1:
        vs = [jnp.minimum(vs[i], vs[i + 1]) for i in range(0, len(vs) - 1, 2)] \
            + ([vs[-1]] if len(vs) % 2 else [])
    return vs[0]


def _tile_kernel(a_ref, br_ref, out1_ref, out2_ref, colacc_ref, *, n_i, m):
    i = pl.program_id(1)

    @pl.when(i == 0)
    def _init():
        colacc_ref[...] = jnp.full((8, m), jnp.inf, jnp.float32)

    for h in range(_TN // _SW):
        hs = h * _SW
        axb = jnp.broadcast_to(a_ref[0, 0, hs:hs + _SW, 0:1], (_SW, 128))
        ayb = jnp.broadcast_to(a_ref[0, 0, hs:hs + _SW, 1:2], (_SW, 128))
        azb = jnp.broadcast_to(a_ref[0, 0, hs:hs + _SW, 2:3], (_SW, 128))
        nab = jnp.broadcast_to(a_ref[0, 0, hs:hs + _SW, 3:4], (_SW, 128))
        ax = [axb[8 * g:8 * (g + 1), :] for g in range(_G)]
        ay = [ayb[8 * g:8 * (g + 1), :] for g in range(_G)]
        az = [azb[8 * g:8 * (g + 1), :] for g in range(_G)]
        na = [nab[8 * g:8 * (g + 1), :] for g in range(_G)]

        inf = jnp.full((8, 128), jnp.inf, jnp.float32)
        rowaccs = [inf] * _G
        for c in range(m // 128):
            cs = 128 * c
            bx = br_ref[0, 0, :, cs:cs + 128]
            by = br_ref[0, 1, :, cs:cs + 128]
            bz = br_ref[0, 2, :, cs:cs + 128]
            nb = br_ref[0, 3, :, cs:cs + 128]
            colf = []
            for g in range(_G):
                e = ax[g] * bx + nb
                e = ay[g] * by + e
                e = az[g] * bz + e
                f = e + na[g]
                colf.append(f)
                rowaccs[g] = jnp.minimum(rowaccs[g], f)
            cm = _tree_min(colf)
            colacc_ref[:, cs:cs + 128] = jnp.minimum(
                colacc_ref[:, cs:cs + 128], cm)

        rowacc = jnp.concatenate(rowaccs, axis=0)
        out1_ref[0, 0, 0, hs:hs + _SW] = jnp.min(rowacc, axis=1)

    @pl.when(i == n_i - 1)
    def _finish():
        out2_ref[0, 0, :] = jnp.min(colacc_ref[...], axis=0)


_NSC = 4096
_NTEC = 32
_RB = 4
_SEG = 2048


def _sc_worker(a_hbm, b_hbm, d1_hbm, colp_hbm, a_v, bseg_v, colacc_v, rowout_v):
    rows = a_v.shape[1]
    m = colacc_v.shape[0]
    seg = bseg_v.shape[1]
    wid = lax.axis_index("s") * 2 + lax.axis_index("c")
    pltpu.sync_copy(a_hbm.at[wid], a_v)

    inf16 = jnp.full((16,), jnp.inf, jnp.float32)

    def initj(j, carry):
        colacc_v[pl.ds(j * 16, 16)] = inf16
        return carry

    lax.fori_loop(0, m // 16, initj, 0)

    def initr(r, carry):
        rowout_v[r, :] = inf16
        return carry

    lax.fori_loop(0, rows, initr, 0)

    def seg_body(sg, carry):
        pltpu.sync_copy(b_hbm.at[wid, sg], bseg_v)
        c0 = sg * seg

        def blk(bi, c2):
            r0 = bi * 16
            axv = a_v[0, pl.ds(r0, 16)]
            ayv = a_v[1, pl.ds(r0, 16)]
            azv = a_v[2, pl.ds(r0, 16)]
            nnv = a_v[3, pl.ds(r0, 16)]
            for sub in range(16 // _RB):
                rowcoef = []
                for k in range(_RB):
                    idx = sub * _RB + k
                    rowcoef.append((jnp.full((16,), axv[idx]),
                                    jnp.full((16,), ayv[idx]),
                                    jnp.full((16,), azv[idx]),
                                    jnp.full((16,), nnv[idx])))

                def jbody(j, accs):
                    ch = pl.ds(j * 16, 16)
                    bx = bseg_v[0, ch]
                    by = bseg_v[1, ch]
                    bz = bseg_v[2, ch]
                    nb = bseg_v[3, ch]
                    gh = pl.ds(c0 + j * 16, 16)
                    fs = []
                    out = []
                    for k in range(_RB):
                        ax, ay, az, nn = rowcoef[k]
                        e = ax * bx + nb
                        e = ay * by + e
                        e = az * bz + e
                        f = e + nn
                        fs.append(f)
                        out.append(jnp.minimum(accs[k], f))
                    cm = _tree_min(fs)
                    colacc_v[gh] = jnp.minimum(colacc_v[gh], cm)
                    return tuple(out)

                accs = lax.fori_loop(0, seg // 16, jbody, (inf16,) * _RB)
                for k in range(_RB):
                    r = r0 + sub * _RB + k
                    rowout_v[r, :] = jnp.minimum(rowout_v[r, :], accs[k])
            return c2

        lax.fori_loop(0, rows // 16, blk, 0)
        return carry

    lax.fori_loop(0, m // seg, seg_body, 0)
    pltpu.sync_copy(rowout_v, d1_hbm.at[wid])
    pltpu.sync_copy(colacc_v, colp_hbm.at[wid])


def _sc_chamfer(a_sc, b_sc, m):
    rows = a_sc.shape[2]
    run = pl.kernel(
        _sc_worker,
        out_type=(
            jax.ShapeDtypeStruct((_NTEC, rows, 16), jnp.float32),
            jax.ShapeDtypeStruct((_NTEC, m), jnp.float32),
        ),
        mesh=plsc.VectorSubcoreMesh(
            core_axis_name="c", subcore_axis_name="s",
            num_cores=2, num_subcores=16),
        scratch_types=[
            pltpu.VMEM((4, rows), jnp.float32),
            pltpu.VMEM((4, _SEG), jnp.float32),
            pltpu.VMEM((m,), jnp.float32),
            pltpu.VMEM((rows, 16), jnp.float32),
        ],
    )
    return run(a_sc, b_sc)


def _chamfer_fused(x1, x2):
    b, n, _ = x1.shape
    m = x2.shape[1]
    n_tc = n - _NSC
    assert n_tc % _TN == 0 and m % 128 == 0
    n_i = n_tc // _TN

    na = jnp.sum(x1 * x1, axis=-1)
    nb = jnp.sum(x2 * x2, axis=-1)
    a_all = jnp.concatenate([-2.0 * x1, na[..., None]], axis=-1)
    a = a_all[:, :n_tc, :].reshape(b, n_i, _TN, 4)
    bt = jnp.concatenate([x2, nb[..., None]], axis=-1).transpose(0, 2, 1)
    br = jnp.broadcast_to(bt[:, :, None, :], (b, 4, 8, m))

    rows = _NSC * b // _NTEC
    a_sc = a_all[:, n_tc:, :].reshape(_NTEC, rows, 4).transpose(0, 2, 1)
    b_sc = jnp.broadcast_to(
        bt[:, None, :, :], (b, _NTEC // b, 4, m)).reshape(_NTEC, 4, m)
    b_sc = b_sc.reshape(_NTEC, 4, m // _SEG, _SEG).transpose(0, 2, 1, 3)
    d1_sc, colpart = _sc_chamfer(a_sc, b_sc, m)

    out1, out2 = pl.pallas_call(
        functools.partial(_tile_kernel, n_i=n_i, m=m),
        grid=(b, n_i),
        in_specs=[
            pl.BlockSpec((1, 1, _TN, 4), lambda bi, i: (bi, i, 0, 0)),
            pl.BlockSpec((1, 4, 8, m), lambda bi, i: (bi, 0, 0, 0)),
        ],
        out_specs=[
            pl.BlockSpec((1, 1, 1, _TN), lambda bi, i: (bi, i, 0, 0)),
            pl.BlockSpec((1, 1, m), lambda bi, i: (bi, 0, 0)),
        ],
        out_shape=[
            jax.ShapeDtypeStruct((b, n_i, 1, _TN), jnp.float32),
            jax.ShapeDtypeStruct((b, 1, m), jnp.float32),
        ],
        scratch_shapes=[pltpu.VMEM((8, m), jnp.float32)],
    )(a, br)

    dist1 = jnp.concatenate(
        [out1.reshape(b, n_tc),
         jnp.min(d1_sc.reshape(b, _NSC, 16), axis=-1)], axis=1)
    dist2 = jnp.minimum(
        out2.reshape(b, m),
        jnp.min(colpart.reshape(b, _NTEC // b, m), axis=1))
    return dist1, dist2


def kernel(xyz1, xyz2):
    dist1, dist2 = _chamfer_fused(xyz1, xyz2)
    return (dist1, dist2)


# speedup vs baseline: 1.2793x; 1.2793x over previous
import functools

import jax
import jax.numpy as jnp
from jax import lax
from jax.experimental import pallas as pl
from jax.experimental.pallas import tpu as pltpu
from jax.experimental.pallas import tpu_sc as plsc

_TN = 256
_SW = 128
_G = _SW // 8


def _tree_min(vs):
    while len(vs) > 1:
        vs = [jnp.minimum(vs[i], vs[i + 1]) for i in range(0, len(vs) - 1, 2)] \
            + ([vs[-1]] if len(vs) % 2 else [])
    return vs[0]


def _tile_kernel(a_ref, br_ref, out1_ref, out2_ref, colacc_ref, *, n_i, m):
    i = pl.program_id(1)

    @pl.when(i == 0)
    def _init():
        colacc_ref[...] = jnp.full((8, m), jnp.inf, jnp.float32)

    for h in range(_TN // _SW):
        hs = h * _SW
        axb = jnp.broadcast_to(a_ref[0, 0, hs:hs + _SW, 0:1], (_SW, 128))
        ayb = jnp.broadcast_to(a_ref[0, 0, hs:hs + _SW, 1:2], (_SW, 128))
        azb = jnp.broadcast_to(a_ref[0, 0, hs:hs + _SW, 2:3], (_SW, 128))
        nab = jnp.broadcast_to(a_ref[0, 0, hs:hs + _SW, 3:4], (_SW, 128))
        ax = [axb[8 * g:8 * (g + 1), :] for g in range(_G)]
        ay = [ayb[8 * g:8 * (g + 1), :] for g in range(_G)]
        az = [azb[8 * g:8 * (g + 1), :] for g in range(_G)]
        na = [nab[8 * g:8 * (g + 1), :] for g in range(_G)]

        inf = jnp.full((8, 128), jnp.inf, jnp.float32)
        rowaccs = [inf] * _G
        for c in range(m // 128):
            cs = 128 * c
            bx = br_ref[0, 0, :, cs:cs + 128]
            by = br_ref[0, 1, :, cs:cs + 128]
            bz = br_ref[0, 2, :, cs:cs + 128]
            nb = br_ref[0, 3, :, cs:cs + 128]
            colf = []
            for g in range(_G):
                e = ax[g] * bx + nb
                e = ay[g] * by + e
                e = az[g] * bz + e
                f = e + na[g]
                colf.append(f)
                rowaccs[g] = jnp.minimum(rowaccs[g], f)
            cm = _tree_min(colf)
            colacc_ref[:, cs:cs + 128] = jnp.minimum(
                colacc_ref[:, cs:cs + 128], cm)

        rowacc = jnp.concatenate(rowaccs, axis=0)
        out1_ref[0, 0, 0, hs:hs + _SW] = jnp.min(rowacc, axis=1)

    @pl.when(i == n_i - 1)
    def _finish():
        out2_ref[0, 0, :] = jnp.min(colacc_ref[...], axis=0)


_NSC = 4096
_NTEC = 32
_RB = 8
_SEG = 2048


def _sc_worker(a_hbm, b_hbm, d1_hbm, colp_hbm, a_v, bseg_v, colacc_v, rowout_v):
    rows = a_v.shape[1]
    m = colacc_v.shape[0]
    seg = bseg_v.shape[1]
    wid = lax.axis_index("s") * 2 + lax.axis_index("c")
    pltpu.sync_copy(a_hbm.at[wid], a_v)

    inf16 = jnp.full((16,), jnp.inf, jnp.float32)

    def initj(j, carry):
        colacc_v[pl.ds(j * 16, 16)] = inf16
        return carry

    lax.fori_loop(0, m // 16, initj, 0)

    def initr(r, carry):
        rowout_v[r, :] = inf16
        return carry

    lax.fori_loop(0, rows, initr, 0)

    def seg_body(sg, carry):
        pltpu.sync_copy(b_hbm.at[wid, sg], bseg_v)
        c0 = sg * seg

        def blk(bi, c2):
            r0 = bi * 16
            axv = a_v[0, pl.ds(r0, 16)]
            ayv = a_v[1, pl.ds(r0, 16)]
            azv = a_v[2, pl.ds(r0, 16)]
            nnv = a_v[3, pl.ds(r0, 16)]
            for sub in range(16 // _RB):
                rowcoef = []
                for k in range(_RB):
                    idx = sub * _RB + k
                    rowcoef.append((jnp.full((16,), axv[idx]),
                                    jnp.full((16,), ayv[idx]),
                                    jnp.full((16,), azv[idx]),
                                    jnp.full((16,), nnv[idx])))

                def jbody(j, accs):
                    ch = pl.ds(j * 16, 16)
                    bx = bseg_v[0, ch]
                    by = bseg_v[1, ch]
                    bz = bseg_v[2, ch]
                    nb = bseg_v[3, ch]
                    gh = pl.ds(c0 + j * 16, 16)
                    fs = []
                    out = []
                    for k in range(_RB):
                        ax, ay, az, nn = rowcoef[k]
                        e = ax * bx + nb
                        e = ay * by + e
                        e = az * bz + e
                        f = e + nn
                        fs.append(f)
                        out.append(jnp.minimum(accs[k], f))
                    cm = _tree_min(fs)
                    colacc_v[gh] = jnp.minimum(colacc_v[gh], cm)
                    return tuple(out)

                accs = lax.fori_loop(0, seg // 16, jbody, (inf16,) * _RB)
                for k in range(_RB):
                    r = r0 + sub * _RB + k
                    rowout_v[r, :] = jnp.minimum(rowout_v[r, :], accs[k])
            return c2

        lax.fori_loop(0, rows // 16, blk, 0)
        return carry

    lax.fori_loop(0, m // seg, seg_body, 0)
    pltpu.sync_copy(rowout_v, d1_hbm.at[wid])
    pltpu.sync_copy(colacc_v, colp_hbm.at[wid])


def _sc_chamfer(a_sc, b_sc, m):
    rows = a_sc.shape[2]
    run = pl.kernel(
        _sc_worker,
        out_type=(
            jax.ShapeDtypeStruct((_NTEC, rows, 16), jnp.float32),
            jax.ShapeDtypeStruct((_NTEC, m), jnp.float32),
        ),
        mesh=plsc.VectorSubcoreMesh(
            core_axis_name="c", subcore_axis_name="s",
            num_cores=2, num_subcores=16),
        scratch_types=[
            pltpu.VMEM((4, rows), jnp.float32),
            pltpu.VMEM((4, _SEG), jnp.float32),
            pltpu.VMEM((m,), jnp.float32),
            pltpu.VMEM((rows, 16), jnp.float32),
        ],
    )
    return run(a_sc, b_sc)


def _chamfer_fused(x1, x2):
    b, n, _ = x1.shape
    m = x2.shape[1]
    n_tc = n - _NSC
    assert n_tc % _TN == 0 and m % 128 == 0
    n_i = n_tc // _TN

    na = jnp.sum(x1 * x1, axis=-1)
    nb = jnp.sum(x2 * x2, axis=-1)
    a_all = jnp.concatenate([-2.0 * x1, na[..., None]], axis=-1)
    a = a_all[:, :n_tc, :].reshape(b, n_i, _TN, 4)
    bt = jnp.concatenate([x2, nb[..., None]], axis=-1).transpose(0, 2, 1)
    br = jnp.broadcast_to(bt[:, :, None, :], (b, 4, 8, m))

    rows = _NSC * b // _NTEC
    a_sc = a_all[:, n_tc:, :].reshape(_NTEC, rows, 4).transpose(0, 2, 1)
    b_sc = jnp.broadcast_to(
        bt[:, None, :, :], (b, _NTEC // b, 4, m)).reshape(_NTEC, 4, m)
    b_sc = b_sc.reshape(_NTEC, 4, m // _SEG, _SEG).transpose(0, 2, 1, 3)
    d1_sc, colpart = _sc_chamfer(a_sc, b_sc, m)

    out1, out2 = pl.pallas_call(
        functools.partial(_tile_kernel, n_i=n_i, m=m),
        grid=(b, n_i),
        in_specs=[
            pl.BlockSpec((1, 1, _TN, 4), lambda bi, i: (bi, i, 0, 0)),
            pl.BlockSpec((1, 4, 8, m), lambda bi, i: (bi, 0, 0, 0)),
        ],
        out_specs=[
            pl.BlockSpec((1, 1, 1, _TN), lambda bi, i: (bi, i, 0, 0)),
            pl.BlockSpec((1, 1, m), lambda bi, i: (bi, 0, 0)),
        ],
        out_shape=[
            jax.ShapeDtypeStruct((b, n_i, 1, _TN), jnp.float32),
            jax.ShapeDtypeStruct((b, 1, m), jnp.float32),
        ],
        scratch_shapes=[pltpu.VMEM((8, m), jnp.float32)],
    )(a, br)

    dist1 = jnp.concatenate(
        [out1.reshape(b, n_tc),
         jnp.min(d1_sc.reshape(b, _NSC, 16), axis=-1)], axis=1)
    dist2 = jnp.minimum(
        out2.reshape(b, m),
        jnp.min(colpart.reshape(b, _NTEC // b, m), axis=1))
    return dist1, dist2


def kernel(xyz1, xyz2):
    dist1, dist2 = _chamfer_fused(xyz1, xyz2)
    return (dist1, dist2)


# speedup vs baseline: 1.4532x; 1.1359x over previous
import functools

import jax
import jax.numpy as jnp
from jax import lax
from jax.experimental import pallas as pl
from jax.experimental.pallas import tpu as pltpu
from jax.experimental.pallas import tpu_sc as plsc

_TN = 256
_SW = 128
_G = _SW // 8


def _tree_min(vs):
    while len(vs) > 1:
        vs = [jnp.minimum(vs[i], vs[i + 1]) for i in range(0, len(vs) - 1, 2)] \
            + ([vs[-1]] if len(vs) % 2 else [])
    return vs[0]


def _tile_kernel(a_ref, br_ref, out1_ref, out2_ref, colacc_ref, *, n_i, m):
    i = pl.program_id(1)

    @pl.when(i == 0)
    def _init():
        colacc_ref[...] = jnp.full((8, m), jnp.inf, jnp.float32)

    for h in range(_TN // _SW):
        hs = h * _SW
        axb = jnp.broadcast_to(a_ref[0, 0, hs:hs + _SW, 0:1], (_SW, 128))
        ayb = jnp.broadcast_to(a_ref[0, 0, hs:hs + _SW, 1:2], (_SW, 128))
        azb = jnp.broadcast_to(a_ref[0, 0, hs:hs + _SW, 2:3], (_SW, 128))
        nab = jnp.broadcast_to(a_ref[0, 0, hs:hs + _SW, 3:4], (_SW, 128))
        ax = [axb[8 * g:8 * (g + 1), :] for g in range(_G)]
        ay = [ayb[8 * g:8 * (g + 1), :] for g in range(_G)]
        az = [azb[8 * g:8 * (g + 1), :] for g in range(_G)]
        na = [nab[8 * g:8 * (g + 1), :] for g in range(_G)]

        inf = jnp.full((8, 128), jnp.inf, jnp.float32)
        rowaccs = [inf] * _G
        for c in range(m // 128):
            cs = 128 * c
            bx = br_ref[0, 0, :, cs:cs + 128]
            by = br_ref[0, 1, :, cs:cs + 128]
            bz = br_ref[0, 2, :, cs:cs + 128]
            nb = br_ref[0, 3, :, cs:cs + 128]
            colf = []
            for g in range(_G):
                e = ax[g] * bx + nb
                e = ay[g] * by + e
                e = az[g] * bz + e
                f = e + na[g]
                colf.append(f)
                rowaccs[g] = jnp.minimum(rowaccs[g], f)
            cm = _tree_min(colf)
            colacc_ref[:, cs:cs + 128] = jnp.minimum(
                colacc_ref[:, cs:cs + 128], cm)

        rowacc = jnp.concatenate(rowaccs, axis=0)
        out1_ref[0, 0, 0, hs:hs + _SW] = jnp.min(rowacc, axis=1)

    @pl.when(i == n_i - 1)
    def _finish():
        out2_ref[0, 0, :] = jnp.min(colacc_ref[...], axis=0)


_NSC = 3584
_NTEC = 32
_RB = 8
_SEG = 2048


def _sc_worker(a_hbm, b_hbm, d1_hbm, colp_hbm, a_v, bseg_v, colacc_v, rowout_v):
    rows = a_v.shape[1]
    m = colacc_v.shape[0]
    seg = bseg_v.shape[1]
    wid = lax.axis_index("s") * 2 + lax.axis_index("c")
    pltpu.sync_copy(a_hbm.at[wid], a_v)

    inf16 = jnp.full((16,), jnp.inf, jnp.float32)

    def initj(j, carry):
        colacc_v[pl.ds(j * 16, 16)] = inf16
        return carry

    lax.fori_loop(0, m // 16, initj, 0)

    def initr(r, carry):
        rowout_v[r, :] = inf16
        return carry

    lax.fori_loop(0, rows, initr, 0)

    def seg_body(sg, carry):
        pltpu.sync_copy(b_hbm.at[wid, sg], bseg_v)
        c0 = sg * seg

        def blk(bi, c2):
            r0 = bi * 16
            axv = a_v[0, pl.ds(r0, 16)]
            ayv = a_v[1, pl.ds(r0, 16)]
            azv = a_v[2, pl.ds(r0, 16)]
            nnv = a_v[3, pl.ds(r0, 16)]
            for sub in range(16 // _RB):
                rowcoef = []
                for k in range(_RB):
                    idx = sub * _RB + k
                    rowcoef.append((jnp.full((16,), axv[idx]),
                                    jnp.full((16,), ayv[idx]),
                                    jnp.full((16,), azv[idx]),
                                    jnp.full((16,), nnv[idx])))

                def jbody(j, accs):
                    ch = pl.ds(j * 16, 16)
                    bx = bseg_v[0, ch]
                    by = bseg_v[1, ch]
                    bz = bseg_v[2, ch]
                    nb = bseg_v[3, ch]
                    gh = pl.ds(c0 + j * 16, 16)
                    fs = []
                    out = []
                    for k in range(_RB):
                        ax, ay, az, nn = rowcoef[k]
                        e = ax * bx + nb
                        e = ay * by + e
                        e = az * bz + e
                        f = e + nn
                        fs.append(f)
                        out.append(jnp.minimum(accs[k], f))
                    cm = _tree_min(fs)
                    colacc_v[gh] = jnp.minimum(colacc_v[gh], cm)
                    return tuple(out)

                accs = lax.fori_loop(0, seg // 16, jbody, (inf16,) * _RB)
                for k in range(_RB):
                    r = r0 + sub * _RB + k
                    rowout_v[r, :] = jnp.minimum(rowout_v[r, :], accs[k])
            return c2

        lax.fori_loop(0, rows // 16, blk, 0)
        return carry

    lax.fori_loop(0, m // seg, seg_body, 0)
    pltpu.sync_copy(rowout_v, d1_hbm.at[wid])
    pltpu.sync_copy(colacc_v, colp_hbm.at[wid])


def _sc_chamfer(a_sc, b_sc, m):
    rows = a_sc.shape[2]
    run = pl.kernel(
        _sc_worker,
        out_type=(
            jax.ShapeDtypeStruct((_NTEC, rows, 16), jnp.float32),
            jax.ShapeDtypeStruct((_NTEC, m), jnp.float32),
        ),
        mesh=plsc.VectorSubcoreMesh(
            core_axis_name="c", subcore_axis_name="s",
            num_cores=2, num_subcores=16),
        scratch_types=[
            pltpu.VMEM((4, rows), jnp.float32),
            pltpu.VMEM((4, _SEG), jnp.float32),
            pltpu.VMEM((m,), jnp.float32),
            pltpu.VMEM((rows, 16), jnp.float32),
        ],
    )
    return run(a_sc, b_sc)


def _chamfer_fused(x1, x2):
    b, n, _ = x1.shape
    m = x2.shape[1]
    n_tc = n - _NSC
    assert n_tc % _TN == 0 and m % 128 == 0
    n_i = n_tc // _TN

    na = jnp.sum(x1 * x1, axis=-1)
    nb = jnp.sum(x2 * x2, axis=-1)
    a_all = jnp.concatenate([-2.0 * x1, na[..., None]], axis=-1)
    a = a_all[:, :n_tc, :].reshape(b, n_i, _TN, 4)
    bt = jnp.concatenate([x2, nb[..., None]], axis=-1).transpose(0, 2, 1)
    br = jnp.broadcast_to(bt[:, :, None, :], (b, 4, 8, m))

    rows = _NSC * b // _NTEC
    a_sc = a_all[:, n_tc:, :].reshape(_NTEC, rows, 4).transpose(0, 2, 1)
    b_sc = jnp.broadcast_to(
        bt[:, None, :, :], (b, _NTEC // b, 4, m)).reshape(_NTEC, 4, m)
    b_sc = b_sc.reshape(_NTEC, 4, m // _SEG, _SEG).transpose(0, 2, 1, 3)
    d1_sc, colpart = _sc_chamfer(a_sc, b_sc, m)

    out1, out2 = pl.pallas_call(
        functools.partial(_tile_kernel, n_i=n_i, m=m),
        grid=(b, n_i),
        in_specs=[
            pl.BlockSpec((1, 1, _TN, 4), lambda bi, i: (bi, i, 0, 0)),
            pl.BlockSpec((1, 4, 8, m), lambda bi, i: (bi, 0, 0, 0)),
        ],
        out_specs=[
            pl.BlockSpec((1, 1, 1, _TN), lambda bi, i: (bi, i, 0, 0)),
            pl.BlockSpec((1, 1, m), lambda bi, i: (bi, 0, 0)),
        ],
        out_shape=[
            jax.ShapeDtypeStruct((b, n_i, 1, _TN), jnp.float32),
            jax.ShapeDtypeStruct((b, 1, m), jnp.float32),
        ],
        scratch_shapes=[pltpu.VMEM((8, m), jnp.float32)],
    )(a, br)

    dist1 = jnp.concatenate(
        [out1.reshape(b, n_tc),
         jnp.min(d1_sc.reshape(b, _NSC, 16), axis=-1)], axis=1)
    dist2 = jnp.minimum(
        out2.reshape(b, m),
        jnp.min(colpart.reshape(b, _NTEC // b, m), axis=1))
    return dist1, dist2


def kernel(xyz1, xyz2):
    dist1, dist2 = _chamfer_fused(xyz1, xyz2)
    return (dist1, dist2)


# speedup vs baseline: 1.4787x; 1.0176x over previous
import functools

import jax
import jax.numpy as jnp
from jax import lax
from jax.experimental import pallas as pl
from jax.experimental.pallas import tpu as pltpu
from jax.experimental.pallas import tpu_sc as plsc

_TN = 256
_SW = 128
_G = _SW // 8


def _tree_min(vs):
    while len(vs) > 1:
        vs = [jnp.minimum(vs[i], vs[i + 1]) for i in range(0, len(vs) - 1, 2)] \
            + ([vs[-1]] if len(vs) % 2 else [])
    return vs[0]


def _tile_kernel(a_ref, br_ref, out1_ref, out2_ref, colacc_ref, *, n_i, m):
    i = pl.program_id(1)

    @pl.when(i == 0)
    def _init():
        colacc_ref[...] = jnp.full((8, m), jnp.inf, jnp.float32)

    for h in range(_TN // _SW):
        hs = h * _SW
        axb = jnp.broadcast_to(a_ref[0, 0, hs:hs + _SW, 0:1], (_SW, 128))
        ayb = jnp.broadcast_to(a_ref[0, 0, hs:hs + _SW, 1:2], (_SW, 128))
        azb = jnp.broadcast_to(a_ref[0, 0, hs:hs + _SW, 2:3], (_SW, 128))
        nab = jnp.broadcast_to(a_ref[0, 0, hs:hs + _SW, 3:4], (_SW, 128))
        ax = [axb[8 * g:8 * (g + 1), :] for g in range(_G)]
        ay = [ayb[8 * g:8 * (g + 1), :] for g in range(_G)]
        az = [azb[8 * g:8 * (g + 1), :] for g in range(_G)]
        na = [nab[8 * g:8 * (g + 1), :] for g in range(_G)]

        inf = jnp.full((8, 128), jnp.inf, jnp.float32)
        rowaccs = [inf] * _G
        for c in range(m // 128):
            cs = 128 * c
            bx = br_ref[0, 0, :, cs:cs + 128]
            by = br_ref[0, 1, :, cs:cs + 128]
            bz = br_ref[0, 2, :, cs:cs + 128]
            nb = br_ref[0, 3, :, cs:cs + 128]
            colf = []
            for g in range(_G):
                e = ax[g] * bx + nb
                e = ay[g] * by + e
                e = az[g] * bz + e
                f = e + na[g]
                colf.append(f)
                rowaccs[g] = jnp.minimum(rowaccs[g], f)
            cm = _tree_min(colf)
            colacc_ref[:, cs:cs + 128] = jnp.minimum(
                colacc_ref[:, cs:cs + 128], cm)

        rowacc = jnp.concatenate(rowaccs, axis=0)
        out1_ref[0, 0, 0, hs:hs + _SW] = jnp.min(rowacc, axis=1)

    @pl.when(i == n_i - 1)
    def _finish():
        out2_ref[0, 0, :] = jnp.min(colacc_ref[...], axis=0)


_NSC = 3584
_NTEC = 32
_RB = 8
_SEG = 2048


def _sc_worker(a_hbm, b_hbm, d1_hbm, colp_hbm, a_v, bseg_v, colacc_v, rowout_v):
    rows = a_v.shape[1]
    m = colacc_v.shape[0]
    seg = bseg_v.shape[1]
    wid = lax.axis_index("s") * 2 + lax.axis_index("c")
    pltpu.sync_copy(a_hbm.at[wid], a_v)

    inf16 = jnp.full((16,), jnp.inf, jnp.float32)

    def initj(j, carry):
        colacc_v[pl.ds(j * 16, 16)] = inf16
        return carry

    lax.fori_loop(0, m // 16, initj, 0)

    def initr(r, carry):
        rowout_v[r, :] = inf16
        return carry

    lax.fori_loop(0, rows, initr, 0)

    def seg_body(sg, carry):
        pltpu.sync_copy(b_hbm.at[wid, sg], bseg_v)
        c0 = sg * seg

        def blk(bi, c2):
            r0 = bi * 16
            axv = a_v[0, pl.ds(r0, 16)]
            ayv = a_v[1, pl.ds(r0, 16)]
            azv = a_v[2, pl.ds(r0, 16)]
            nnv = a_v[3, pl.ds(r0, 16)]
            for sub in range(16 // _RB):
                rowcoef = []
                for k in range(_RB):
                    idx = sub * _RB + k
                    rowcoef.append((jnp.full((16,), axv[idx]),
                                    jnp.full((16,), ayv[idx]),
                                    jnp.full((16,), azv[idx]),
                                    jnp.full((16,), nnv[idx])))

                def jbody(j, accs):
                    ch = pl.ds(j * 16, 16)
                    bx = bseg_v[0, ch]
                    by = bseg_v[1, ch]
                    bz = bseg_v[2, ch]
                    nb = bseg_v[3, ch]
                    gh = pl.ds(c0 + j * 16, 16)
                    fs = []
                    out = []
                    for k in range(_RB):
                        ax, ay, az, nn = rowcoef[k]
                        e = ax * bx + nb
                        e = ay * by + e
                        e = az * bz + e
                        f = e + nn
                        fs.append(f)
                        out.append(jnp.minimum(accs[k], f))
                    cm = _tree_min(fs)
                    colacc_v[gh] = jnp.minimum(colacc_v[gh], cm)
                    return tuple(out)

                accs = lax.fori_loop(0, seg // 16, jbody, (inf16,) * _RB,
                                     unroll=2)
                for k in range(_RB):
                    r = r0 + sub * _RB + k
                    rowout_v[r, :] = jnp.minimum(rowout_v[r, :], accs[k])
            return c2

        lax.fori_loop(0, rows // 16, blk, 0)
        return carry

    lax.fori_loop(0, m // seg, seg_body, 0)
    pltpu.sync_copy(rowout_v, d1_hbm.at[wid])
    pltpu.sync_copy(colacc_v, colp_hbm.at[wid])


def _sc_chamfer(a_sc, b_sc, m):
    rows = a_sc.shape[2]
    run = pl.kernel(
        _sc_worker,
        out_type=(
            jax.ShapeDtypeStruct((_NTEC, rows, 16), jnp.float32),
            jax.ShapeDtypeStruct((_NTEC, m), jnp.float32),
        ),
        mesh=plsc.VectorSubcoreMesh(
            core_axis_name="c", subcore_axis_name="s",
            num_cores=2, num_subcores=16),
        scratch_types=[
            pltpu.VMEM((4, rows), jnp.float32),
            pltpu.VMEM((4, _SEG), jnp.float32),
            pltpu.VMEM((m,), jnp.float32),
            pltpu.VMEM((rows, 16), jnp.float32),
        ],
    )
    return run(a_sc, b_sc)


def _chamfer_fused(x1, x2):
    b, n, _ = x1.shape
    m = x2.shape[1]
    n_tc = n - _NSC
    assert n_tc % _TN == 0 and m % 128 == 0
    n_i = n_tc // _TN

    na = jnp.sum(x1 * x1, axis=-1)
    nb = jnp.sum(x2 * x2, axis=-1)
    a_all = jnp.concatenate([-2.0 * x1, na[..., None]], axis=-1)
    a = a_all[:, :n_tc, :].reshape(b, n_i, _TN, 4)
    bt = jnp.concatenate([x2, nb[..., None]], axis=-1).transpose(0, 2, 1)
    br = jnp.broadcast_to(bt[:, :, None, :], (b, 4, 8, m))

    rows = _NSC * b // _NTEC
    a_sc = a_all[:, n_tc:, :].reshape(_NTEC, rows, 4).transpose(0, 2, 1)
    b_sc = jnp.broadcast_to(
        bt[:, None, :, :], (b, _NTEC // b, 4, m)).reshape(_NTEC, 4, m)
    b_sc = b_sc.reshape(_NTEC, 4, m // _SEG, _SEG).transpose(0, 2, 1, 3)
    d1_sc, colpart = _sc_chamfer(a_sc, b_sc, m)

    out1, out2 = pl.pallas_call(
        functools.partial(_tile_kernel, n_i=n_i, m=m),
        grid=(b, n_i),
        in_specs=[
            pl.BlockSpec((1, 1, _TN, 4), lambda bi, i: (bi, i, 0, 0)),
            pl.BlockSpec((1, 4, 8, m), lambda bi, i: (bi, 0, 0, 0)),
        ],
        out_specs=[
            pl.BlockSpec((1, 1, 1, _TN), lambda bi, i: (bi, i, 0, 0)),
            pl.BlockSpec((1, 1, m), lambda bi, i: (bi, 0, 0)),
        ],
        out_shape=[
            jax.ShapeDtypeStruct((b, n_i, 1, _TN), jnp.float32),
            jax.ShapeDtypeStruct((b, 1, m), jnp.float32),
        ],
        scratch_shapes=[pltpu.VMEM((8, m), jnp.float32)],
    )(a, br)

    dist1 = jnp.concatenate(
        [out1.reshape(b, n_tc),
         jnp.min(d1_sc.reshape(b, _NSC, 16), axis=-1)], axis=1)
    dist2 = jnp.minimum(
        out2.reshape(b, m),
        jnp.min(colpart.reshape(b, _NTEC // b, m), axis=1))
    return dist1, dist2


def kernel(xyz1, xyz2):
    dist1, dist2 = _chamfer_fused(xyz1, xyz2)
    return (dist1, dist2)
